# Initial kernel scaffold; baseline (speedup 1.0000x reference)
import jax
import jax.numpy as jnp
from jax.experimental import pallas as pl


def kernel(x, rbf, neighbors, W1, b1, Wf1, bf1, Wf2, bf2, W2, b2):
    raise NotImplementedError("write your pallas kernel here")


# speedup vs baseline: 1.1744x; 1.1744x over previous
import functools

import jax
import jax.numpy as jnp
from jax import lax
from jax.experimental import pallas as pl
from jax.experimental.pallas import tpu as pltpu
from jax.experimental.pallas import tpu_sc as plsc

N, K, F, R = 10000, 32, 128, 128
E = N * K

NC, NS = 2, 16
NW = NC * NS

CHUNK = 128
CPW = (-(-E // (CHUNK * NW)) + 7) // 8 * 8
NCHUNK = CPW * NW
EPAD = NCHUNK * CHUNK


def _softplus(z):
    return jnp.maximum(z, 0.0) + jnp.log1p(jnp.exp(-jnp.abs(z)))


def _v_body(x_ref, w_ref, b_ref, o_ref):
    z = jnp.dot(x_ref[...], w_ref[...], preferred_element_type=jnp.float32)
    o_ref[...] = _softplus(z + b_ref[...])


def _atom_layer1(x, W1, b1):
    blk = 2000
    return pl.pallas_call(
        _v_body,
        grid=(N // blk,),
        in_specs=[
            pl.BlockSpec((blk, F), lambda i: (i, 0)),
            pl.BlockSpec((F, F), lambda i: (0, 0)),
            pl.BlockSpec((1, F), lambda i: (0, 0)),
        ],
        out_specs=pl.BlockSpec((blk, F), lambda i: (i, 0)),
        out_shape=jax.ShapeDtypeStruct((N, F), jnp.float32),
    )(x, W1, b1.reshape(1, F))


def _sc_gather_body(table_hbm, idx_hbm, out_hbm, idx_v, rows_v, gsem):
    wid = lax.axis_index("s") * NC + lax.axis_index("c")
    base = wid * CPW
    pltpu.sync_copy(idx_hbm.at[pl.ds(base, CPW)], idx_v)

    def body(j, carry):
        pltpu.async_copy(table_hbm.at[idx_v.at[j]], rows_v, gsem).wait()
        pltpu.sync_copy(rows_v, out_hbm.at[pl.ds((base + j) * CHUNK, CHUNK)])
        return carry

    lax.fori_loop(0, CPW, body, 0)


@functools.cache
def _make_sc_gather():
    return pl.kernel(
        _sc_gather_body,
        out_type=jax.ShapeDtypeStruct((EPAD, F), jnp.float32),
        mesh=plsc.VectorSubcoreMesh(core_axis_name="c", subcore_axis_name="s"),
        scratch_types=[
            pltpu.VMEM((CPW, CHUNK), jnp.int32),
            pltpu.VMEM((CHUNK, F), jnp.float32),
            pltpu.SemaphoreType.DMA,
        ],
    )


BN = 400
BE = BN * K


def _main_body(rbf_ref, nbr_ref, x_ref, wf1_ref, bf1_ref, wf2_ref, bf2_ref,
               w2_ref, b2_ref, o_ref):
    h = _softplus(
        jnp.dot(rbf_ref[...], wf1_ref[...], preferred_element_type=jnp.float32)
        + bf1_ref[...])
    filt = (jnp.dot(h, wf2_ref[...], preferred_element_type=jnp.float32)
            + bf2_ref[...])
    prod = filt * nbr_ref[...]
    agg = prod.reshape(BN, K, F).sum(axis=1)
    v2 = _softplus(
        jnp.dot(agg, w2_ref[...], preferred_element_type=jnp.float32)
        + b2_ref[...])
    o_ref[...] = x_ref[...] + v2


def _cfconv_out(rbf_flat, nbr, x, Wf1, bf1, Wf2, bf2, W2, b2):
    return pl.pallas_call(
        _main_body,
        grid=(N // BN,),
        in_specs=[
            pl.BlockSpec((BE, R), lambda i: (i, 0)),
            pl.BlockSpec((BE, F), lambda i: (i, 0)),
            pl.BlockSpec((BN, F), lambda i: (i, 0)),
            pl.BlockSpec((R, F), lambda i: (0, 0)),
            pl.BlockSpec((1, F), lambda i: (0, 0)),
            pl.BlockSpec((F, F), lambda i: (0, 0)),
            pl.BlockSpec((1, F), lambda i: (0, 0)),
            pl.BlockSpec((F, F), lambda i: (0, 0)),
            pl.BlockSpec((1, F), lambda i: (0, 0)),
        ],
        out_specs=pl.BlockSpec((BN, F), lambda i: (i, 0)),
        out_shape=jax.ShapeDtypeStruct((N, F), jnp.float32),
    )(rbf_flat, nbr, x, Wf1, bf1.reshape(1, F), Wf2, bf2.reshape(1, F),
      W2, b2.reshape(1, F))


@jax.jit
def kernel(x, rbf, neighbors, W1, b1, Wf1, bf1, Wf2, bf2, W2, b2):
    v = _atom_layer1(x, W1, b1)
    idx = jnp.concatenate(
        [neighbors.reshape(-1),
         jnp.zeros((EPAD - E,), jnp.int32)]).reshape(NCHUNK, CHUNK)
    nbr = _make_sc_gather()(v, idx)
    return _cfconv_out(rbf.reshape(E, R), nbr, x, Wf1, bf1, Wf2, bf2, W2, b2)


# speedup vs baseline: 1.2610x; 1.0738x over previous
import functools

import jax
import jax.numpy as jnp
from jax import lax
from jax.experimental import pallas as pl
from jax.experimental.pallas import tpu as pltpu
from jax.experimental.pallas import tpu_sc as plsc

N, K, F, R = 10000, 32, 128, 128
E = N * K

NC, NS = 2, 16
NW = NC * NS

CHUNK = 128
CPW = (-(-E // (CHUNK * NW)) + 7) // 8 * 8
NCHUNK = CPW * NW
EPAD = NCHUNK * CHUNK


def _softplus(z):
    return jnp.maximum(z, 0.0) + jnp.log1p(jnp.exp(-jnp.abs(z)))


def _v_body(x_ref, w_ref, b_ref, o_ref):
    z = jnp.dot(x_ref[...], w_ref[...], preferred_element_type=jnp.float32)
    o_ref[...] = _softplus(z + b_ref[...])


def _atom_layer1(x, W1, b1):
    blk = 2000
    return pl.pallas_call(
        _v_body,
        grid=(N // blk,),
        in_specs=[
            pl.BlockSpec((blk, F), lambda i: (i, 0)),
            pl.BlockSpec((F, F), lambda i: (0, 0)),
            pl.BlockSpec((1, F), lambda i: (0, 0)),
        ],
        out_specs=pl.BlockSpec((blk, F), lambda i: (i, 0)),
        out_shape=jax.ShapeDtypeStruct((N, F), jnp.float32),
    )(x, W1, b1.reshape(1, F))


NBUF = 4
NG = CPW // NBUF


def _sc_gather_body(table_hbm, idx_hbm, out_hbm, idx_v, rows_v, gsem, ssem):
    wid = lax.axis_index("s") * NC + lax.axis_index("c")
    base = wid * CPW
    pltpu.sync_copy(idx_hbm.at[pl.ds(base, CPW)], idx_v)

    def group(g, carry):
        @pl.when(g > 0)
        def _():
            for b in range(NBUF):
                pltpu.make_async_copy(
                    rows_v.at[b],
                    out_hbm.at[pl.ds((base + (g - 1) * NBUF + b) * CHUNK,
                                     CHUNK)],
                    ssem).wait()

        for b in range(NBUF):
            pltpu.async_copy(table_hbm.at[idx_v.at[g * NBUF + b]],
                             rows_v.at[b], gsem)
        for b in range(NBUF):
            pltpu.make_async_copy(table_hbm.at[idx_v.at[g * NBUF + b]],
                                  rows_v.at[b], gsem).wait()
        for b in range(NBUF):
            pltpu.async_copy(
                rows_v.at[b],
                out_hbm.at[pl.ds((base + g * NBUF + b) * CHUNK, CHUNK)],
                ssem)
        return carry

    lax.fori_loop(0, NG, group, 0)
    for b in range(NBUF):
        pltpu.make_async_copy(
            rows_v.at[b],
            out_hbm.at[pl.ds((base + (NG - 1) * NBUF + b) * CHUNK, CHUNK)],
            ssem).wait()


@functools.cache
def _make_sc_gather():
    return pl.kernel(
        _sc_gather_body,
        out_type=jax.ShapeDtypeStruct((EPAD, F), jnp.float32),
        mesh=plsc.VectorSubcoreMesh(core_axis_name="c", subcore_axis_name="s"),
        scratch_types=[
            pltpu.VMEM((CPW, CHUNK), jnp.int32),
            pltpu.VMEM((NBUF, CHUNK, F), jnp.float32),
            pltpu.SemaphoreType.DMA,
            pltpu.SemaphoreType.DMA,
        ],
    )


BN = 400
BE = BN * K


def _main_body(rbf_ref, nbr_ref, x_ref, wf1_ref, bf1_ref, wf2_ref, bf2_ref,
               w2_ref, b2_ref, o_ref):
    h = _softplus(
        jnp.dot(rbf_ref[...], wf1_ref[...], preferred_element_type=jnp.float32)
        + bf1_ref[...])
    filt = (jnp.dot(h, wf2_ref[...], preferred_element_type=jnp.float32)
            + bf2_ref[...])
    prod = filt * nbr_ref[...]
    agg = prod.reshape(BN, K, F).sum(axis=1)
    v2 = _softplus(
        jnp.dot(agg, w2_ref[...], preferred_element_type=jnp.float32)
        + b2_ref[...])
    o_ref[...] = x_ref[...] + v2


def _cfconv_out(rbf_flat, nbr, x, Wf1, bf1, Wf2, bf2, W2, b2):
    return pl.pallas_call(
        _main_body,
        grid=(N // BN,),
        in_specs=[
            pl.BlockSpec((BE, R), lambda i: (i, 0)),
            pl.BlockSpec((BE, F), lambda i: (i, 0)),
            pl.BlockSpec((BN, F), lambda i: (i, 0)),
            pl.BlockSpec((R, F), lambda i: (0, 0)),
            pl.BlockSpec((1, F), lambda i: (0, 0)),
            pl.BlockSpec((F, F), lambda i: (0, 0)),
            pl.BlockSpec((1, F), lambda i: (0, 0)),
            pl.BlockSpec((F, F), lambda i: (0, 0)),
            pl.BlockSpec((1, F), lambda i: (0, 0)),
        ],
        out_specs=pl.BlockSpec((BN, F), lambda i: (i, 0)),
        out_shape=jax.ShapeDtypeStruct((N, F), jnp.float32),
    )(rbf_flat, nbr, x, Wf1, bf1.reshape(1, F), Wf2, bf2.reshape(1, F),
      W2, b2.reshape(1, F))


@jax.jit
def kernel(x, rbf, neighbors, W1, b1, Wf1, bf1, Wf2, bf2, W2, b2):
    v = _atom_layer1(x, W1, b1)
    idx = jnp.concatenate(
        [neighbors.reshape(-1),
         jnp.zeros((EPAD - E,), jnp.int32)]).reshape(NCHUNK, CHUNK)
    nbr = _make_sc_gather()(v, idx)
    return _cfconv_out(rbf.reshape(E, R), nbr, x, Wf1, bf1, Wf2, bf2, W2, b2)
